# baseline (speedup 1.0000x reference)
import functools

import jax
import jax.numpy as jnp
from jax import lax
from jax.experimental import pallas as pl
from jax.experimental.pallas import tpu as pltpu
from jax.experimental.pallas import tpu_sc as plsc

VOCAB = 100000
SEQ = 50
BATCH = 4096
DIM = 128
TRAIN = 16384

_PAD = 56
_NC = 2
_NS = 16
_NW = _NC * _NS
_QPW = BATCH // _NW
_QG = 2

@functools.cache
def _make_sc_gather_sum():
    mesh = plsc.VectorSubcoreMesh(core_axis_name="c", subcore_axis_name="s")

    @functools.partial(
        pl.kernel,
        mesh=mesh,
        out_type=jax.ShapeDtypeStruct((BATCH, DIM), jnp.float32),
        scratch_types=[
            pltpu.VMEM((_QG * _PAD,), jnp.int32),
            pltpu.VMEM((_QG * _PAD, DIM), jnp.float32),
            pltpu.VMEM((_QG, DIM), jnp.float32),
            pltpu.SemaphoreType.DMA,
        ],
    )
    def _sc_gather_sum(table_hbm, idx_hbm, out_hbm, idx_v, rows_v, out_v, sem):
        wid = lax.axis_index("s") * _NC + lax.axis_index("c")
        qbase = wid * _QPW

        def step(ci, carry):
            q0 = qbase + ci * _QG
            pltpu.sync_copy(idx_hbm.at[pl.ds(q0 * _PAD, _QG * _PAD)], idx_v)
            pltpu.async_copy(table_hbm.at[idx_v], rows_v, sem).wait()
            for q in range(_QG):
                r0 = q * _PAD

                def seqstep(s, acc, r0=r0):
                    return tuple(
                        acc[d] + rows_v[r0 + s, pl.ds(d * 16, 16)] for d in range(8)
                    )

                init = tuple(rows_v[r0, pl.ds(d * 16, 16)] for d in range(8))
                acc = lax.fori_loop(1, SEQ, seqstep, init)
                for d in range(8):
                    out_v[q, pl.ds(d * 16, 16)] = acc[d]
            pltpu.sync_copy(out_v, out_hbm.at[pl.ds(q0, _QG)])
            return carry

        lax.fori_loop(0, _QPW // _QG, step, 0)

    return _sc_gather_sum


_NB = 1024


def _norm_body(t_ref, o_ref):
    t = t_ref[...]
    n = jnp.sqrt(jnp.sum(t * t, axis=1, keepdims=True))
    o_ref[...] = t / jnp.maximum(n, 1e-8)


_normalize_rows = pl.pallas_call(
    _norm_body,
    grid=(TRAIN // _NB,),
    in_specs=[pl.BlockSpec((_NB, DIM), lambda i: (i, 0))],
    out_specs=pl.BlockSpec((_NB, DIM), lambda i: (i, 0)),
    out_shape=jax.ShapeDtypeStruct((TRAIN, DIM), jnp.float32),
)


_BT = 512
_CT = 2048


def _knn_body(q_ref, t_ref, y_ref, yp_ref, sc_ref):
    q = q_ref[...] / jnp.float32(SEQ)
    qn = q / jnp.maximum(jnp.sqrt(jnp.sum(q * q, axis=1, keepdims=True)), 1e-8)

    def step(c, carry):
        best, bidx = carry
        t = t_ref[pl.ds(c * _CT, _CT), :]
        s = lax.dot_general(
            qn, t, (((1,), (1,)), ((), ())), preferred_element_type=jnp.float32
        )
        m = jnp.max(s, axis=1, keepdims=True)
        ii = lax.broadcasted_iota(jnp.int32, (_BT, _CT), 1)
        cand = jnp.where(s == m, ii, jnp.int32(1 << 30))
        a = jnp.min(cand, axis=1, keepdims=True) + c * _CT
        upd = m > best
        return jnp.where(upd, m, best), jnp.where(upd, a, bidx)

    best, bidx = lax.fori_loop(
        0,
        TRAIN // _CT,
        step,
        (
            jnp.full((_BT, 1), -jnp.inf, jnp.float32),
            jnp.zeros((_BT, 1), jnp.int32),
        ),
    )

    def lstep(c, lab):
        y = y_ref[0, pl.ds(c * _CT, _CT)]
        ii = lax.broadcasted_iota(jnp.int32, (_BT, _CT), 1) + c * _CT
        hit = ii == bidx
        pick = jnp.where(hit, jnp.broadcast_to(y[None, :], (_BT, _CT)), 0)
        return lab + jnp.sum(pick, axis=1, keepdims=True)

    y_pred = lax.fori_loop(0, TRAIN // _CT, lstep, jnp.zeros((_BT, 1), jnp.int32))
    yp_ref[...] = y_pred
    sc_ref[...] = best


_knn = pl.pallas_call(
    _knn_body,
    grid=(BATCH // _BT,),
    in_specs=[
        pl.BlockSpec((_BT, DIM), lambda b: (b, 0)),
        pl.BlockSpec((TRAIN, DIM), lambda b: (0, 0)),
        pl.BlockSpec((1, TRAIN), lambda b: (0, 0)),
    ],
    out_specs=[
        pl.BlockSpec((_BT, 1), lambda b: (b, 0)),
        pl.BlockSpec((_BT, 1), lambda b: (b, 0)),
    ],
    out_shape=[
        jax.ShapeDtypeStruct((BATCH, 1), jnp.int32),
        jax.ShapeDtypeStruct((BATCH, 1), jnp.float32),
    ],
)


def kernel(insts, emb_table, train_embs, y_train):
    idx_pad = jnp.pad(insts.astype(jnp.int32).T, ((0, 0), (0, _PAD - SEQ)))
    idx_flat = idx_pad.reshape(-1)
    sums = _make_sc_gather_sum()(emb_table, idx_flat)
    tn = _normalize_rows(train_embs)
    y2d = y_train.astype(jnp.int32).reshape(1, TRAIN)
    yp2d, sc2d = _knn(sums, tn, y2d)
    return yp2d.reshape(BATCH), sc2d.reshape(BATCH)


# speedup vs baseline: 1.0027x; 1.0027x over previous
import functools

import jax
import jax.numpy as jnp
from jax import lax
from jax.experimental import pallas as pl
from jax.experimental.pallas import tpu as pltpu
from jax.experimental.pallas import tpu_sc as plsc

VOCAB = 100000
SEQ = 50
BATCH = 4096
DIM = 128
TRAIN = 16384

_PAD = 56
_NC = 2
_NS = 16
_NW = _NC * _NS
_QPW = BATCH // _NW
_QG = 2

_NBUF = 4
_NG = _QPW // _QG


@functools.cache
def _make_sc_gather_sum():
    mesh = plsc.VectorSubcoreMesh(core_axis_name="c", subcore_axis_name="s")

    @functools.partial(
        pl.kernel,
        mesh=mesh,
        out_type=jax.ShapeDtypeStruct((BATCH, DIM), jnp.float32),
        scratch_types=[
            pltpu.VMEM((_QPW * _PAD,), jnp.int32),
            pltpu.VMEM((_QPW, DIM), jnp.float32),
        ]
        + [pltpu.VMEM((_QG * _PAD, DIM), jnp.float32) for _ in range(_NBUF)]
        + [pltpu.SemaphoreType.DMA for _ in range(_NBUF)],
    )
    def _sc_gather_sum(table_hbm, idx_hbm, out_hbm, idx_v, out_v, *bufs):
        rows = bufs[:_NBUF]
        sems = bufs[_NBUF:]
        wid = lax.axis_index("s") * _NC + lax.axis_index("c")
        qbase = wid * _QPW
        pltpu.sync_copy(idx_hbm.at[pl.ds(qbase * _PAD, _QPW * _PAD)], idx_v)

        def gsrc(g):
            return table_hbm.at[idx_v.at[pl.ds(g * _QG * _PAD, _QG * _PAD)]]

        for b in range(_NBUF):
            pltpu.async_copy(gsrc(b), rows[b], sems[b])

        def outer(i, carry):
            for b in range(_NBUF):
                g = i * _NBUF + b
                pltpu.make_async_copy(gsrc(g), rows[b], sems[b]).wait()
                for q in range(_QG):
                    r0 = q * _PAD

                    def seqstep(s, acc, b=b, r0=r0):
                        return tuple(
                            acc[d] + rows[b][r0 + s, pl.ds(d * 16, 16)]
                            for d in range(8)
                        )

                    init = tuple(rows[b][r0, pl.ds(d * 16, 16)] for d in range(8))
                    acc = lax.fori_loop(1, SEQ, seqstep, init, unroll=7)
                    orow = g * _QG + q
                    for d in range(8):
                        out_v[orow, pl.ds(d * 16, 16)] = acc[d]
                nxt = g + _NBUF

                @pl.when(nxt < _NG)
                def _(b=b, nxt=nxt):
                    pltpu.async_copy(gsrc(nxt), rows[b], sems[b])

            return carry

        lax.fori_loop(0, _NG // _NBUF, outer, 0)
        pltpu.sync_copy(out_v, out_hbm.at[pl.ds(qbase, _QPW)])

    return _sc_gather_sum


_NB = 1024


def _norm_body(t_ref, o_ref):
    t = t_ref[...]
    n = jnp.sqrt(jnp.sum(t * t, axis=1, keepdims=True))
    o_ref[...] = t / jnp.maximum(n, 1e-8)


_normalize_rows = pl.pallas_call(
    _norm_body,
    grid=(TRAIN // _NB,),
    in_specs=[pl.BlockSpec((_NB, DIM), lambda i: (i, 0))],
    out_specs=pl.BlockSpec((_NB, DIM), lambda i: (i, 0)),
    out_shape=jax.ShapeDtypeStruct((TRAIN, DIM), jnp.float32),
)


_BT = 512
_CT = 2048


def _knn_body(q_ref, t_ref, y_ref, yp_ref, sc_ref):
    q = q_ref[...] / jnp.float32(SEQ)
    qn = q / jnp.maximum(jnp.sqrt(jnp.sum(q * q, axis=1, keepdims=True)), 1e-8)

    def step(c, carry):
        best, bidx = carry
        t = t_ref[pl.ds(c * _CT, _CT), :]
        s = lax.dot_general(
            qn, t, (((1,), (1,)), ((), ())), preferred_element_type=jnp.float32
        )
        m = jnp.max(s, axis=1, keepdims=True)
        ii = lax.broadcasted_iota(jnp.int32, (_BT, _CT), 1)
        cand = jnp.where(s == m, ii, jnp.int32(1 << 30))
        a = jnp.min(cand, axis=1, keepdims=True) + c * _CT
        upd = m > best
        return jnp.where(upd, m, best), jnp.where(upd, a, bidx)

    best, bidx = lax.fori_loop(
        0,
        TRAIN // _CT,
        step,
        (
            jnp.full((_BT, 1), -jnp.inf, jnp.float32),
            jnp.zeros((_BT, 1), jnp.int32),
        ),
    )

    def lstep(c, lab):
        y = y_ref[0, pl.ds(c * _CT, _CT)]
        ii = lax.broadcasted_iota(jnp.int32, (_BT, _CT), 1) + c * _CT
        hit = ii == bidx
        pick = jnp.where(hit, jnp.broadcast_to(y[None, :], (_BT, _CT)), 0)
        return lab + jnp.sum(pick, axis=1, keepdims=True)

    y_pred = lax.fori_loop(0, TRAIN // _CT, lstep, jnp.zeros((_BT, 1), jnp.int32))
    yp_ref[...] = y_pred
    sc_ref[...] = best


_knn = pl.pallas_call(
    _knn_body,
    grid=(BATCH // _BT,),
    in_specs=[
        pl.BlockSpec((_BT, DIM), lambda b: (b, 0)),
        pl.BlockSpec((TRAIN, DIM), lambda b: (0, 0)),
        pl.BlockSpec((1, TRAIN), lambda b: (0, 0)),
    ],
    out_specs=[
        pl.BlockSpec((_BT, 1), lambda b: (b, 0)),
        pl.BlockSpec((_BT, 1), lambda b: (b, 0)),
    ],
    out_shape=[
        jax.ShapeDtypeStruct((BATCH, 1), jnp.int32),
        jax.ShapeDtypeStruct((BATCH, 1), jnp.float32),
    ],
)


def kernel(insts, emb_table, train_embs, y_train):
    idx_pad = jnp.pad(insts.astype(jnp.int32).T, ((0, 0), (0, _PAD - SEQ)))
    idx_flat = idx_pad.reshape(-1)
    sums = _make_sc_gather_sum()(emb_table, idx_flat)
    tn = _normalize_rows(train_embs)
    y2d = y_train.astype(jnp.int32).reshape(1, TRAIN)
    yp2d, sc2d = _knn(sums, tn, y2d)
    return yp2d.reshape(BATCH), sc2d.reshape(BATCH)


# speedup vs baseline: 5.4747x; 5.4602x over previous
import functools

import jax
import jax.numpy as jnp
from jax import lax
from jax.experimental import pallas as pl
from jax.experimental.pallas import tpu as pltpu
from jax.experimental.pallas import tpu_sc as plsc

VOCAB = 100000
SEQ = 50
BATCH = 4096
DIM = 128
TRAIN = 16384

_PAD = 56
_NC = 2
_NS = 16
_NW = _NC * _NS
_QPW = BATCH // _NW
_QG = 2

_NBUF = 4
_NG = _QPW // _QG


@functools.cache
def _make_sc_gather_sum():
    mesh = plsc.VectorSubcoreMesh(core_axis_name="c", subcore_axis_name="s")

    @functools.partial(
        pl.kernel,
        mesh=mesh,
        out_type=jax.ShapeDtypeStruct((BATCH, DIM), jnp.float32),
        scratch_types=[
            pltpu.VMEM((_QPW * _PAD,), jnp.int32),
            pltpu.VMEM((_QPW, DIM), jnp.float32),
        ]
        + [pltpu.VMEM((_QG * _PAD, DIM), jnp.float32) for _ in range(_NBUF)]
        + [pltpu.SemaphoreType.DMA for _ in range(_NBUF)],
    )
    def _sc_gather_sum(table_hbm, idx_hbm, out_hbm, idx_v, out_v, *bufs):
        rows = bufs[:_NBUF]
        sems = bufs[_NBUF:]
        wid = lax.axis_index("s") * _NC + lax.axis_index("c")
        qbase = wid * _QPW
        pltpu.sync_copy(idx_hbm.at[pl.ds(qbase * _PAD, _QPW * _PAD)], idx_v)

        def gsrc(g):
            return table_hbm.at[idx_v.at[pl.ds(g * _QG * _PAD, _QG * _PAD)]]

        for b in range(_NBUF):
            pltpu.async_copy(gsrc(b), rows[b], sems[b])

        def outer(i, carry):
            for b in range(_NBUF):
                g = i * _NBUF + b
                pltpu.make_async_copy(gsrc(g), rows[b], sems[b]).wait()
                for q in range(_QG):
                    r0 = q * _PAD

                    def seqstep(s, acc, b=b, r0=r0):
                        return tuple(
                            acc[d] + rows[b][r0 + s, pl.ds(d * 16, 16)]
                            for d in range(8)
                        )

                    init = tuple(rows[b][r0, pl.ds(d * 16, 16)] for d in range(8))
                    acc = lax.fori_loop(1, SEQ, seqstep, init, unroll=7)
                    orow = g * _QG + q
                    for d in range(8):
                        out_v[orow, pl.ds(d * 16, 16)] = acc[d]
                nxt = g + _NBUF

                @pl.when(nxt < _NG)
                def _(b=b, nxt=nxt):
                    pltpu.async_copy(gsrc(nxt), rows[b], sems[b])

            return carry

        lax.fori_loop(0, _NG // _NBUF, outer, 0)
        pltpu.sync_copy(out_v, out_hbm.at[pl.ds(qbase, _QPW)])

    return _sc_gather_sum


_NB = 1024


def _norm_body(t_ref, o_ref):
    t = t_ref[...]
    n = jnp.sqrt(jnp.sum(t * t, axis=1, keepdims=True))
    o_ref[...] = t / jnp.maximum(n, 1e-8)


_normalize_rows = pl.pallas_call(
    _norm_body,
    grid=(TRAIN // _NB,),
    in_specs=[pl.BlockSpec((_NB, DIM), lambda i: (i, 0))],
    out_specs=pl.BlockSpec((_NB, DIM), lambda i: (i, 0)),
    out_shape=jax.ShapeDtypeStruct((TRAIN, DIM), jnp.float32),
)


_BT = 512
_CT = 2048


def _knn_body(q_ref, t_ref, y_ref, yp_ref, sc_ref):
    q = q_ref[...] / jnp.float32(SEQ)
    qn = q / jnp.maximum(jnp.sqrt(jnp.sum(q * q, axis=1, keepdims=True)), 1e-8)

    def step(c, carry):
        best, bidx = carry
        t = t_ref[pl.ds(c * _CT, _CT), :]
        s = lax.dot_general(
            qn, t, (((1,), (1,)), ((), ())), preferred_element_type=jnp.float32
        )
        m = jnp.max(s, axis=1, keepdims=True)
        ii = lax.broadcasted_iota(jnp.int32, (_BT, _CT), 1)
        cand = jnp.where(s == m, ii, jnp.int32(1 << 30))
        a = jnp.min(cand, axis=1, keepdims=True) + c * _CT
        upd = m > best
        return jnp.where(upd, m, best), jnp.where(upd, a, bidx)

    best, bidx = lax.fori_loop(
        0,
        TRAIN // _CT,
        step,
        (
            jnp.full((_BT, 1), -jnp.inf, jnp.float32),
            jnp.zeros((_BT, 1), jnp.int32),
        ),
    )

    def lstep(c, lab):
        y = y_ref[0, pl.ds(c * _CT, _CT)]
        ii = lax.broadcasted_iota(jnp.int32, (_BT, _CT), 1) + c * _CT
        hit = ii == bidx
        pick = jnp.where(hit, jnp.broadcast_to(y[None, :], (_BT, _CT)), 0)
        return lab + jnp.sum(pick, axis=1, keepdims=True)

    y_pred = lax.fori_loop(0, TRAIN // _CT, lstep, jnp.zeros((_BT, 1), jnp.int32))
    yp_ref[...] = y_pred
    sc_ref[...] = best


_knn = pl.pallas_call(
    _knn_body,
    grid=(BATCH // _BT,),
    in_specs=[
        pl.BlockSpec((_BT, DIM), lambda b: (b, 0)),
        pl.BlockSpec((TRAIN, DIM), lambda b: (0, 0)),
        pl.BlockSpec((1, TRAIN), lambda b: (0, 0)),
    ],
    out_specs=[
        pl.BlockSpec((_BT, 1), lambda b: (b, 0)),
        pl.BlockSpec((_BT, 1), lambda b: (b, 0)),
    ],
    out_shape=[
        jax.ShapeDtypeStruct((BATCH, 1), jnp.int32),
        jax.ShapeDtypeStruct((BATCH, 1), jnp.float32),
    ],
)


def kernel(insts, emb_table, train_embs, y_train):
    npad = _PAD - SEQ
    pad = (jnp.arange(BATCH, dtype=jnp.int32)[:, None] * npad
           + jnp.arange(npad, dtype=jnp.int32)[None, :]) % VOCAB
    idx_pad = jnp.concatenate([insts.astype(jnp.int32).T, pad], axis=1)
    idx_flat = idx_pad.reshape(-1)
    sums = _make_sc_gather_sum()(emb_table, idx_flat)
    tn = _normalize_rows(train_embs)
    y2d = y_train.astype(jnp.int32).reshape(1, TRAIN)
    yp2d, sc2d = _knn(sums, tn, y2d)
    return yp2d.reshape(BATCH), sc2d.reshape(BATCH)


# speedup vs baseline: 7.6420x; 1.3959x over previous
import functools

import jax
import jax.numpy as jnp
from jax import lax
from jax.experimental import pallas as pl
from jax.experimental.pallas import tpu as pltpu
from jax.experimental.pallas import tpu_sc as plsc

VOCAB = 100000
SEQ = 50
BATCH = 4096
DIM = 128
TRAIN = 16384

_PAD = 56
_NC = 2
_NS = 16
_NW = _NC * _NS
_QPW = BATCH // _NW
_QG = 2

_NBUF = 4
_NG = _QPW // _QG


@functools.cache
def _make_sc_gather_sum():
    mesh = plsc.VectorSubcoreMesh(core_axis_name="c", subcore_axis_name="s")

    @functools.partial(
        pl.kernel,
        mesh=mesh,
        out_type=jax.ShapeDtypeStruct((BATCH, DIM), jnp.float32),
        scratch_types=[
            pltpu.VMEM((_QPW * _PAD,), jnp.int32),
            pltpu.VMEM((_QPW, DIM), jnp.float32),
        ]
        + [pltpu.VMEM((_QG * _PAD, DIM), jnp.float32) for _ in range(_NBUF)]
        + [pltpu.SemaphoreType.DMA for _ in range(_NBUF)],
    )
    def _sc_gather_sum(table_hbm, idx_hbm, out_hbm, idx_v, out_v, *bufs):
        rows = bufs[:_NBUF]
        sems = bufs[_NBUF:]
        wid = lax.axis_index("s") * _NC + lax.axis_index("c")
        qbase = wid * _QPW
        pltpu.sync_copy(idx_hbm.at[pl.ds(qbase * _PAD, _QPW * _PAD)], idx_v)

        def gsrc(g):
            return table_hbm.at[idx_v.at[pl.ds(g * _QG * _PAD, _QG * _PAD)]]

        for b in range(_NBUF):
            pltpu.async_copy(gsrc(b), rows[b], sems[b])

        def outer(i, carry):
            for b in range(_NBUF):
                g = i * _NBUF + b
                pltpu.make_async_copy(gsrc(g), rows[b], sems[b]).wait()
                for q in range(_QG):
                    r0 = q * _PAD

                    def seqstep(s, acc, b=b, r0=r0):
                        return tuple(
                            acc[d] + rows[b][r0 + s, pl.ds(d * 16, 16)]
                            for d in range(8)
                        )

                    init = tuple(rows[b][r0, pl.ds(d * 16, 16)] for d in range(8))
                    acc = lax.fori_loop(1, SEQ, seqstep, init, unroll=7)
                    orow = g * _QG + q
                    for d in range(8):
                        out_v[orow, pl.ds(d * 16, 16)] = acc[d]
                nxt = g + _NBUF

                @pl.when(nxt < _NG)
                def _(b=b, nxt=nxt):
                    pltpu.async_copy(gsrc(nxt), rows[b], sems[b])

            return carry

        lax.fori_loop(0, _NG // _NBUF, outer, 0)
        pltpu.sync_copy(out_v, out_hbm.at[pl.ds(qbase, _QPW)])

    return _sc_gather_sum


_NB = 1024


def _norm_body(t_ref, o_ref):
    t = t_ref[...]
    n = jnp.sqrt(jnp.sum(t * t, axis=1, keepdims=True))
    o_ref[...] = t / jnp.maximum(n, 1e-8)


_normalize_rows = pl.pallas_call(
    _norm_body,
    grid=(TRAIN // _NB,),
    in_specs=[pl.BlockSpec((_NB, DIM), lambda i: (i, 0))],
    out_specs=pl.BlockSpec((_NB, DIM), lambda i: (i, 0)),
    out_shape=jax.ShapeDtypeStruct((TRAIN, DIM), jnp.float32),
)


_BT = 512
_CT = 4096


def _knn_body(q_ref, t_ref, y_ref, yp_ref, sc_ref):
    q = q_ref[...] / jnp.float32(SEQ)
    qn = q / jnp.maximum(jnp.sqrt(jnp.sum(q * q, axis=1, keepdims=True)), 1e-8)

    def step(c, carry):
        best, bkey = carry
        t = t_ref[pl.ds(c * _CT, _CT), :]
        s = lax.dot_general(
            qn, t, (((1,), (1,)), ((), ())), preferred_element_type=jnp.float32
        )
        m = jnp.max(s, axis=1, keepdims=True)
        ii = lax.broadcasted_iota(jnp.int32, (1, _CT), 1) + c * _CT
        keyrow = ((ii << 5) | y_ref[0:1, pl.ds(c * _CT, _CT)]).astype(jnp.float32)
        cand = jnp.where(s == m, jnp.broadcast_to(keyrow, (_BT, _CT)), 1e9)
        k = jnp.min(cand, axis=1, keepdims=True)
        upd = m > best
        return jnp.where(upd, m, best), jnp.where(upd, k, bkey)

    best, bkey = lax.fori_loop(
        0,
        TRAIN // _CT,
        step,
        (
            jnp.full((_BT, 1), -jnp.inf, jnp.float32),
            jnp.zeros((_BT, 1), jnp.float32),
        ),
    )
    yp_ref[...] = bkey.astype(jnp.int32) & 31
    sc_ref[...] = best


_knn = pl.pallas_call(
    _knn_body,
    grid=(BATCH // _BT,),
    in_specs=[
        pl.BlockSpec((_BT, DIM), lambda b: (b, 0)),
        pl.BlockSpec((TRAIN, DIM), lambda b: (0, 0)),
        pl.BlockSpec((1, TRAIN), lambda b: (0, 0)),
    ],
    out_specs=[
        pl.BlockSpec((_BT, 1), lambda b: (b, 0)),
        pl.BlockSpec((_BT, 1), lambda b: (b, 0)),
    ],
    out_shape=[
        jax.ShapeDtypeStruct((BATCH, 1), jnp.int32),
        jax.ShapeDtypeStruct((BATCH, 1), jnp.float32),
    ],
)


def kernel(insts, emb_table, train_embs, y_train):
    npad = _PAD - SEQ
    pad = (jnp.arange(BATCH, dtype=jnp.int32)[:, None] * npad
           + jnp.arange(npad, dtype=jnp.int32)[None, :]) % VOCAB
    idx_pad = jnp.concatenate([insts.astype(jnp.int32).T, pad], axis=1)
    idx_flat = idx_pad.reshape(-1)
    sums = _make_sc_gather_sum()(emb_table, idx_flat)
    tn = _normalize_rows(train_embs)
    y2d = y_train.astype(jnp.int32).reshape(1, TRAIN)
    yp2d, sc2d = _knn(sums, tn, y2d)
    return yp2d.reshape(BATCH), sc2d.reshape(BATCH)


# speedup vs baseline: 8.3360x; 1.0908x over previous
import functools

import jax
import jax.numpy as jnp
from jax import lax
from jax.experimental import pallas as pl
from jax.experimental.pallas import tpu as pltpu
from jax.experimental.pallas import tpu_sc as plsc

VOCAB = 100000
SEQ = 50
BATCH = 4096
DIM = 128
TRAIN = 16384

_PAD = 56
_NC = 2
_NS = 16
_NW = _NC * _NS
_QG = 2

_NBUF = 4


@functools.cache
def _make_sc_gather_sum(nbatch):
    qpw = nbatch // _NW
    ng = qpw // _QG
    mesh = plsc.VectorSubcoreMesh(core_axis_name="c", subcore_axis_name="s")

    @functools.partial(
        pl.kernel,
        mesh=mesh,
        out_type=jax.ShapeDtypeStruct((nbatch, DIM), jnp.float32),
        scratch_types=[
            pltpu.VMEM((qpw * _PAD,), jnp.int32),
            pltpu.VMEM((qpw, DIM), jnp.float32),
        ]
        + [pltpu.VMEM((_QG * _PAD, DIM), jnp.float32) for _ in range(_NBUF)]
        + [pltpu.SemaphoreType.DMA for _ in range(_NBUF)],
    )
    def _sc_gather_sum(table_hbm, idx_hbm, out_hbm, idx_v, out_v, *bufs):
        rows = bufs[:_NBUF]
        sems = bufs[_NBUF:]
        wid = lax.axis_index("s") * _NC + lax.axis_index("c")
        qbase = wid * qpw
        pltpu.sync_copy(idx_hbm.at[pl.ds(qbase * _PAD, qpw * _PAD)], idx_v)

        def gsrc(g):
            return table_hbm.at[idx_v.at[pl.ds(g * _QG * _PAD, _QG * _PAD)]]

        for b in range(_NBUF):
            pltpu.async_copy(gsrc(b), rows[b], sems[b])

        def outer(i, carry):
            for b in range(_NBUF):
                g = i * _NBUF + b
                pltpu.make_async_copy(gsrc(g), rows[b], sems[b]).wait()
                for q in range(_QG):
                    r0 = q * _PAD

                    def seqstep(s, acc, b=b, r0=r0):
                        return tuple(
                            acc[d] + rows[b][r0 + s, pl.ds(d * 16, 16)]
                            for d in range(8)
                        )

                    init = tuple(rows[b][r0, pl.ds(d * 16, 16)] for d in range(8))
                    acc = lax.fori_loop(1, SEQ, seqstep, init, unroll=7)
                    orow = g * _QG + q
                    for d in range(8):
                        out_v[orow, pl.ds(d * 16, 16)] = acc[d]
                nxt = g + _NBUF

                @pl.when(nxt < ng)
                def _(b=b, nxt=nxt):
                    pltpu.async_copy(gsrc(nxt), rows[b], sems[b])

            return carry

        lax.fori_loop(0, ng // _NBUF, outer, 0)
        pltpu.sync_copy(out_v, out_hbm.at[pl.ds(qbase, qpw)])

    return _sc_gather_sum


_NB = 1024


def _norm_body(t_ref, o_ref):
    t = t_ref[...]
    n = jnp.sqrt(jnp.sum(t * t, axis=1, keepdims=True))
    o_ref[...] = t / jnp.maximum(n, 1e-8)


_normalize_rows = pl.pallas_call(
    _norm_body,
    grid=(TRAIN // _NB,),
    in_specs=[pl.BlockSpec((_NB, DIM), lambda i: (i, 0))],
    out_specs=pl.BlockSpec((_NB, DIM), lambda i: (i, 0)),
    out_shape=jax.ShapeDtypeStruct((TRAIN, DIM), jnp.float32),
)


_BT = 512
_CT = 4096


def _knn_body(q_ref, t_ref, y_ref, yp_ref, sc_ref):
    q = q_ref[...] / jnp.float32(SEQ)
    qn = q / jnp.maximum(jnp.sqrt(jnp.sum(q * q, axis=1, keepdims=True)), 1e-8)

    def step(c, carry):
        best, bkey = carry
        t = t_ref[pl.ds(c * _CT, _CT), :]
        s = lax.dot_general(
            qn, t, (((1,), (1,)), ((), ())), preferred_element_type=jnp.float32
        )
        m = jnp.max(s, axis=1, keepdims=True)
        ii = lax.broadcasted_iota(jnp.int32, (1, _CT), 1) + c * _CT
        keyrow = ((ii << 5) | y_ref[0:1, pl.ds(c * _CT, _CT)]).astype(jnp.float32)
        cand = jnp.where(s == m, jnp.broadcast_to(keyrow, (_BT, _CT)), 1e9)
        k = jnp.min(cand, axis=1, keepdims=True)
        upd = m > best
        return jnp.where(upd, m, best), jnp.where(upd, k, bkey)

    best, bkey = lax.fori_loop(
        0,
        TRAIN // _CT,
        step,
        (
            jnp.full((_BT, 1), -jnp.inf, jnp.float32),
            jnp.zeros((_BT, 1), jnp.float32),
        ),
    )
    yp_ref[...] = bkey.astype(jnp.int32) & 31
    sc_ref[...] = best


@functools.cache
def _make_knn(nbatch):
    return pl.pallas_call(
        _knn_body,
        grid=(nbatch // _BT,),
        in_specs=[
            pl.BlockSpec((_BT, DIM), lambda b: (b, 0)),
            pl.BlockSpec((TRAIN, DIM), lambda b: (0, 0)),
            pl.BlockSpec((1, TRAIN), lambda b: (0, 0)),
        ],
        out_specs=[
            pl.BlockSpec((_BT, 1), lambda b: (b, 0)),
            pl.BlockSpec((_BT, 1), lambda b: (b, 0)),
        ],
        out_shape=[
            jax.ShapeDtypeStruct((nbatch, 1), jnp.int32),
            jax.ShapeDtypeStruct((nbatch, 1), jnp.float32),
        ],
    )


_NSPLIT = 2


def kernel(insts, emb_table, train_embs, y_train):
    npad = _PAD - SEQ
    pad = (jnp.arange(BATCH, dtype=jnp.int32)[:, None] * npad
           + jnp.arange(npad, dtype=jnp.int32)[None, :]) % VOCAB
    idx_pad = jnp.concatenate([insts.astype(jnp.int32).T, pad], axis=1)
    tn = _normalize_rows(train_embs)
    y2d = y_train.astype(jnp.int32).reshape(1, TRAIN)
    nb = BATCH // _NSPLIT
    sc_gather = _make_sc_gather_sum(nb)
    knn = _make_knn(nb)
    sums = [
        sc_gather(emb_table, idx_pad[h * nb:(h + 1) * nb].reshape(-1))
        for h in range(_NSPLIT)
    ]
    outs = [knn(s, tn, y2d) for s in sums]
    yp = jnp.concatenate([o[0] for o in outs], axis=0)
    sc = jnp.concatenate([o[1] for o in outs], axis=0)
    return yp.reshape(BATCH), sc.reshape(BATCH)


# speedup vs baseline: 8.8025x; 1.0560x over previous
import functools

import jax
import jax.numpy as jnp
from jax import lax
from jax.experimental import pallas as pl
from jax.experimental.pallas import tpu as pltpu
from jax.experimental.pallas import tpu_sc as plsc

VOCAB = 100000
SEQ = 50
BATCH = 4096
DIM = 128
TRAIN = 16384

_PAD = 56
_NC = 2
_NS = 16
_NW = _NC * _NS
_QG = 2

_NBUF = 4


@functools.cache
def _make_sc_gather_sum(nbatch):
    qpw = nbatch // _NW
    ng = qpw // _QG
    mesh = plsc.VectorSubcoreMesh(core_axis_name="c", subcore_axis_name="s")

    @functools.partial(
        pl.kernel,
        mesh=mesh,
        out_type=jax.ShapeDtypeStruct((nbatch, DIM), jnp.float32),
        scratch_types=[
            pltpu.VMEM((qpw * _PAD,), jnp.int32),
            pltpu.VMEM((qpw, DIM), jnp.float32),
        ]
        + [pltpu.VMEM((_QG * _PAD, DIM), jnp.float32) for _ in range(_NBUF)]
        + [pltpu.SemaphoreType.DMA for _ in range(_NBUF)],
    )
    def _sc_gather_sum(table_hbm, idx_hbm, out_hbm, idx_v, out_v, *bufs):
        rows = bufs[:_NBUF]
        sems = bufs[_NBUF:]
        wid = lax.axis_index("s") * _NC + lax.axis_index("c")
        qbase = wid * qpw
        pltpu.sync_copy(idx_hbm.at[pl.ds(qbase * _PAD, qpw * _PAD)], idx_v)

        def gsrc(g):
            return table_hbm.at[idx_v.at[pl.ds(g * _QG * _PAD, _QG * _PAD)]]

        for b in range(_NBUF):
            pltpu.async_copy(gsrc(b), rows[b], sems[b])

        def outer(i, carry):
            for b in range(_NBUF):
                g = i * _NBUF + b
                pltpu.make_async_copy(gsrc(g), rows[b], sems[b]).wait()
                for q in range(_QG):
                    r0 = q * _PAD

                    def seqstep(s, acc, b=b, r0=r0):
                        return tuple(
                            acc[d] + rows[b][r0 + s, pl.ds(d * 16, 16)]
                            for d in range(8)
                        )

                    init = tuple(rows[b][r0, pl.ds(d * 16, 16)] for d in range(8))
                    acc = lax.fori_loop(1, SEQ, seqstep, init, unroll=7)
                    orow = g * _QG + q
                    for d in range(8):
                        out_v[orow, pl.ds(d * 16, 16)] = acc[d]
                nxt = g + _NBUF

                @pl.when(nxt < ng)
                def _(b=b, nxt=nxt):
                    pltpu.async_copy(gsrc(nxt), rows[b], sems[b])

            return carry

        lax.fori_loop(0, ng // _NBUF, outer, 0)
        pltpu.sync_copy(out_v, out_hbm.at[pl.ds(qbase, qpw)])

    return _sc_gather_sum


_NB = 1024


def _norm_body(t_ref, o_ref):
    t = t_ref[...]
    n = jnp.sqrt(jnp.sum(t * t, axis=1, keepdims=True))
    o_ref[...] = t / jnp.maximum(n, 1e-8)


_normalize_rows = pl.pallas_call(
    _norm_body,
    grid=(TRAIN // _NB,),
    in_specs=[pl.BlockSpec((_NB, DIM), lambda i: (i, 0))],
    out_specs=pl.BlockSpec((_NB, DIM), lambda i: (i, 0)),
    out_shape=jax.ShapeDtypeStruct((TRAIN, DIM), jnp.float32),
)


_BT = 512
_CT = 4096


def _knn_body(q_ref, t_ref, y_ref, yp_ref, sc_ref):
    q = q_ref[...] / jnp.float32(SEQ)
    qn = q / jnp.maximum(jnp.sqrt(jnp.sum(q * q, axis=1, keepdims=True)), 1e-8)
    s = lax.dot_general(
        qn, t_ref[...], (((1,), (1,)), ((), ())),
        preferred_element_type=jnp.float32,
    )
    m = jnp.max(s, axis=1, keepdims=True)
    ii = lax.broadcasted_iota(jnp.int32, (1, TRAIN), 1)
    keyrow = ((ii << 5) | y_ref[...]).astype(jnp.float32)
    cand = jnp.where(s == m, jnp.broadcast_to(keyrow, (_BT, TRAIN)), 1e9)
    k = jnp.min(cand, axis=1, keepdims=True)
    yp_ref[...] = k.astype(jnp.int32) & 31
    sc_ref[...] = m


@functools.cache
def _make_knn(nbatch):
    return pl.pallas_call(
        _knn_body,
        grid=(nbatch // _BT,),
        in_specs=[
            pl.BlockSpec((_BT, DIM), lambda b: (b, 0)),
            pl.BlockSpec((TRAIN, DIM), lambda b: (0, 0)),
            pl.BlockSpec((1, TRAIN), lambda b: (0, 0)),
        ],
        out_specs=[
            pl.BlockSpec((_BT, 1), lambda b: (b, 0)),
            pl.BlockSpec((_BT, 1), lambda b: (b, 0)),
        ],
        out_shape=[
            jax.ShapeDtypeStruct((nbatch, 1), jnp.int32),
            jax.ShapeDtypeStruct((nbatch, 1), jnp.float32),
        ],
    )


_NSPLIT = 2


def kernel(insts, emb_table, train_embs, y_train):
    npad = _PAD - SEQ
    pad = (jnp.arange(BATCH, dtype=jnp.int32)[:, None] * npad
           + jnp.arange(npad, dtype=jnp.int32)[None, :]) % VOCAB
    idx_pad = jnp.concatenate([insts.astype(jnp.int32).T, pad], axis=1)
    tn = _normalize_rows(train_embs)
    y2d = y_train.astype(jnp.int32).reshape(1, TRAIN)
    nb = BATCH // _NSPLIT
    sc_gather = _make_sc_gather_sum(nb)
    knn = _make_knn(nb)
    sums = [
        sc_gather(emb_table, idx_pad[h * nb:(h + 1) * nb].reshape(-1))
        for h in range(_NSPLIT)
    ]
    outs = [knn(s, tn, y2d) for s in sums]
    yp = jnp.concatenate([o[0] for o in outs], axis=0)
    sc = jnp.concatenate([o[1] for o in outs], axis=0)
    return yp.reshape(BATCH), sc.reshape(BATCH)
